# Initial kernel scaffold; baseline (speedup 1.0000x reference)
import jax
import jax.numpy as jnp
from jax.experimental import pallas as pl


def kernel(g, q, p, create_graph, Wq, bq, Wp, bp, W1, b1, W2, b2):
    raise NotImplementedError("write your pallas kernel here")


# speedup vs baseline: 5.4040x; 5.4040x over previous
import functools

import jax
import jax.numpy as jnp
from jax import lax
from jax.experimental import pallas as pl
from jax.experimental.pallas import tpu as pltpu
from jax.experimental.pallas import tpu_sc as plsc

N = 10000
D = 128
NC = 2
NS = 16
LANES = 16
C = 128
N_PAD = 10240
RPT = N_PAD // NS
ZR = 16

@functools.cache
def _mesh():
  return plsc.VectorSubcoreMesh(
      core_axis_name="c", subcore_axis_name="s", num_cores=NC, num_subcores=NS)


def _fill(ref, rows, cols, value):
  v = jnp.full((LANES,), value, jnp.float32)
  for r in range(rows):
    for j in range(cols // LANES):
      if rows == 1:
        ref[pl.ds(j * LANES, LANES)] = v
      else:
        ref[r, pl.ds(j * LANES, LANES)] = v


def _deg_body(nchunk, et, src_hbm, dst_hbm, deg_hbm, idx_v, ones_v, zb_v,
              sh_deg):
  c = lax.axis_index("c")
  s = lax.axis_index("s")
  _fill(ones_v, 1, C, 1.0)
  _fill(zb_v, 1, RPT, 0.0)
  pltpu.sync_copy(zb_v, sh_deg.at[pl.ds(s * RPT, RPT)])
  plsc.subcore_barrier()

  def chunk(k, ref):
    off = s * et + k * C
    pltpu.sync_copy(ref.at[pl.ds(off, C)], idx_v)
    pltpu.sync_copy(ones_v, sh_deg.at[idx_v], add=True)

  @pl.when(c == 0)
  def _():
    lax.fori_loop(0, nchunk, lambda k, x: (chunk(k, src_hbm), x)[1], 0)

  @pl.when(c == 1)
  def _():
    lax.fori_loop(0, nchunk, lambda k, x: (chunk(k, dst_hbm), x)[1], 0)

  plsc.subcore_barrier()
  pltpu.sync_copy(sh_deg.at[pl.ds(s * RPT, RPT)],
                  deg_hbm.at[c, pl.ds(s * RPT, RPT)])


def _scat_body(nchunk, et, x_hbm, y_hbm, gi_hbm, si_hbm, ox_hbm, oy_hbm,
               gi_v, si_v, rows_v, zrow_v, acc_sh, sem):
  c = lax.axis_index("c")
  s = lax.axis_index("s")
  _fill(zrow_v, ZR, D, 0.0)

  def zloop(t, x):
    pltpu.sync_copy(zrow_v, acc_sh.at[pl.ds(s * RPT + t * ZR, ZR)])
    return x

  lax.fori_loop(0, RPT // ZR, zloop, 0)
  plsc.subcore_barrier()

  def chunk(k, src_ref):
    off = s * et + k * C
    pltpu.sync_copy(gi_hbm.at[pl.ds(off, C)], gi_v)
    pltpu.sync_copy(si_hbm.at[pl.ds(off, C)], si_v)
    pltpu.async_copy(src_ref.at[gi_v], rows_v, sem).wait()
    pltpu.sync_copy(rows_v, acc_sh.at[si_v], add=True)

  @pl.when(c == 0)
  def _():
    lax.fori_loop(0, nchunk, lambda k, x: (chunk(k, x_hbm), x)[1], 0)

  @pl.when(c == 1)
  def _():
    lax.fori_loop(0, nchunk, lambda k, x: (chunk(k, y_hbm), x)[1], 0)

  plsc.subcore_barrier()

  @pl.when(c == 0)
  def _():
    pltpu.sync_copy(acc_sh.at[pl.ds(s * RPT, RPT)],
                    ox_hbm.at[pl.ds(s * RPT, RPT)])

  @pl.when(c == 1)
  def _():
    pltpu.sync_copy(acc_sh.at[pl.ds(s * RPT, RPT)],
                    oy_hbm.at[pl.ds(s * RPT, RPT)])


def _sc_degrees(src_pad, dst_pad, nchunk, et):
  return pl.kernel(
      functools.partial(_deg_body, nchunk, et),
      out_type=jax.ShapeDtypeStruct((NC, N_PAD), jnp.float32),
      mesh=_mesh(),
      scratch_types=[
          pltpu.VMEM((C,), jnp.int32),
          pltpu.VMEM((C,), jnp.float32),
          pltpu.VMEM((RPT,), jnp.float32),
          pltpu.VMEM_SHARED((N_PAD,), jnp.float32),
      ],
  )(src_pad, dst_pad)


def _sc_scatter(x, y, gidx, sidx, nchunk, et):
  return pl.kernel(
      functools.partial(_scat_body, nchunk, et),
      out_type=(jax.ShapeDtypeStruct((N_PAD, D), jnp.float32),
                jax.ShapeDtypeStruct((N_PAD, D), jnp.float32)),
      mesh=_mesh(),
      scratch_types=[
          pltpu.VMEM((C,), jnp.int32),
          pltpu.VMEM((C,), jnp.int32),
          pltpu.VMEM((C, D), jnp.float32),
          pltpu.VMEM((ZR, D), jnp.float32),
          pltpu.VMEM_SHARED((N_PAD, D), jnp.float32),
          pltpu.SemaphoreType.DMA,
      ],
  )(x, y, gidx, sidx)


BR = 1024
GRID = N_PAD // BR


def _scale_body(deg_ref, q_ref, p_ref, qs_ref, ps_ref):
  r = lax.rsqrt(jnp.maximum(deg_ref[...], 1.0))
  qs_ref[...] = q_ref[...] * r
  ps_ref[...] = p_ref[...] * r


def _dense_body(deg_ref, aq_ref, ap_ref, Wq_, Wp_, W1a_, W1b_, W2_, WqT_,
                WpT_, W1aT_, W1bT_, W2T_, bq_, bp_, b1_, b2_, Bq_ref, Bp_ref):
  r = lax.rsqrt(jnp.maximum(deg_ref[...], 1.0))
  dot = functools.partial(jnp.dot, preferred_element_type=jnp.float32)
  qg = dot(aq_ref[...] * r, Wq_[...]) + bq_[...]
  pg = dot(ap_ref[...] * r, Wp_[...]) + bp_[...]
  h1 = jnp.tanh(dot(qg, W1a_[...]) + dot(pg, W1b_[...]) + b1_[...])
  h2 = dot(h1, W2_[...]) + b2_[...]
  G = dot(h2, W2T_[...]) * (1.0 - h1 * h1)
  Bq_ref[...] = dot(dot(G, W1aT_[...]), WqT_[...]) * r
  Bp_ref[...] = dot(dot(G, W1bT_[...]), WpT_[...]) * r


def _final_body(deg_ref, q_ref, p_ref, cq_ref, cp_ref, qn_ref, pn_ref):
  r = lax.rsqrt(jnp.maximum(deg_ref[...], 1.0))
  qn_ref[...] = q_ref[...] + cp_ref[...] * r
  pn_ref[...] = p_ref[...] - cq_ref[...] * r


def _col_spec():
  return pl.BlockSpec((BR, 1), lambda i: (i, 0))


def _row_spec():
  return pl.BlockSpec((BR, D), lambda i: (i, 0))


def _w_spec(shape):
  return pl.BlockSpec(shape, lambda i: (0, 0))


def kernel(g, q, p, create_graph, Wq, bq, Wp, bp, W1, b1, W2, b2):
  del create_graph
  E = g.shape[1]
  et_raw = -(-E // NS)
  nchunk = -(-et_raw // C)
  et = nchunk * C
  e_pad = et * NS
  src = g[0].astype(jnp.int32)
  dst = g[1].astype(jnp.int32)
  npad_idx = e_pad - E
  if npad_idx:
    pad_idx = N + jnp.arange(npad_idx, dtype=jnp.int32) % (N_PAD - N)
    src = jnp.concatenate([src, pad_idx])
    dst = jnp.concatenate([dst, pad_idx])

  degs = _sc_degrees(src, dst, nchunk, et)
  out_deg = degs[0].reshape(N_PAD, 1)
  in_deg = degs[1].reshape(N_PAD, 1)

  qs, ps = pl.pallas_call(
      _scale_body,
      grid=(GRID,),
      in_specs=[_col_spec(), _row_spec(), _row_spec()],
      out_specs=[_row_spec(), _row_spec()],
      out_shape=[jax.ShapeDtypeStruct((N_PAD, D), jnp.float32)] * 2,
  )(out_deg, q, p)

  aggq, aggp = _sc_scatter(qs, ps, src, dst, nchunk, et)

  W1a, W1b = W1[:D], W1[D:]
  Bq, Bp = pl.pallas_call(
      _dense_body,
      grid=(GRID,),
      in_specs=[_col_spec(), _row_spec(), _row_spec()]
      + [_w_spec((D, D))] * 5
      + [_w_spec((D, D))] * 5
      + [_w_spec((1, D))] * 4,
      out_specs=[_row_spec(), _row_spec()],
      out_shape=[jax.ShapeDtypeStruct((N_PAD, D), jnp.float32)] * 2,
  )(in_deg, aggq, aggp, Wq, Wp, W1a, W1b, W2, Wq.T, Wp.T, W1a.T, W1b.T, W2.T,
    bq.reshape(1, D), bp.reshape(1, D), b1.reshape(1, D), b2.reshape(1, D))

  Cq, Cp = _sc_scatter(Bq, Bp, dst, src, nchunk, et)

  q_next, p_next = pl.pallas_call(
      _final_body,
      grid=(GRID,),
      in_specs=[_col_spec()] + [_row_spec()] * 4,
      out_specs=[_row_spec(), _row_spec()],
      out_shape=[jax.ShapeDtypeStruct((N, D), jnp.float32)] * 2,
  )(out_deg, q, p, Cq, Cp)

  return (q_next, p_next)


# speedup vs baseline: 11.8137x; 2.1861x over previous
import functools

import jax
import jax.numpy as jnp
from jax import lax
from jax.experimental import pallas as pl
from jax.experimental.pallas import tpu as pltpu
from jax.experimental.pallas import tpu_sc as plsc

N = 10000
D = 128
NC = 2
NS = 16
LANES = 16
C = 128
GRP = 32
N_PAD = 10240
RPT = N_PAD // NS
ZR = 16
DEG_GRP = 8


@functools.cache
def _mesh():
  return plsc.VectorSubcoreMesh(
      core_axis_name="c", subcore_axis_name="s", num_cores=NC, num_subcores=NS)


def _fill(ref, rows, cols, value):
  v = jnp.full((LANES,), value, jnp.float32)
  for r in range(rows):
    for j in range(cols // LANES):
      if rows == 1:
        ref[pl.ds(j * LANES, LANES)] = v
      else:
        ref[r, pl.ds(j * LANES, LANES)] = v


def _deg_body(nchunk, g2_hbm, deg_hbm, idx_big, ones_v, zb_v, sh_deg,
              semz, semsc):
  c = lax.axis_index("c")
  s = lax.axis_index("s")
  _fill(ones_v, 1, C, 1.0)
  _fill(zb_v, 1, RPT, 0.0)
  pltpu.async_copy(zb_v, sh_deg.at[pl.ds(s * RPT, RPT)], semz)
  pltpu.sync_copy(g2_hbm.at[c, pl.ds(s * nchunk, nchunk)], idx_big)
  pltpu.make_async_copy(zb_v, sh_deg.at[pl.ds(s * RPT, RPT)], semz).wait()
  plsc.subcore_barrier()

  def fire(t):
    for b in range(DEG_GRP):
      pltpu.async_copy(ones_v, sh_deg.at[idx_big.at[t * DEG_GRP + b, 0]],
                       semsc, add=True)

  def drain():
    for _ in range(DEG_GRP):
      pltpu.make_async_copy(ones_v, sh_deg.at[idx_big.at[0, 0]], semsc).wait()

  fire(0)

  def body(t, carry):
    fire(t)
    drain()
    return carry

  lax.fori_loop(1, nchunk // DEG_GRP, body, 0)
  drain()
  plsc.subcore_barrier()
  pltpu.sync_copy(sh_deg.at[pl.ds(s * RPT, RPT)],
                  deg_hbm.at[c, pl.ds(s * RPT, RPT)])


def _scat_body(nchunk, dirn, xy_hbm, g2_hbm, oxy_hbm,
               gi_grp, si_grp, r0, r1, zrow, acc_sh,
               semz, sg0, sg1, ss0, ss1):
  rows = [r0, r1]
  semg = [sg0, sg1]
  sems = [ss0, ss1]
  c = lax.axis_index("c")
  s = lax.axis_index("s")
  _fill(zrow, ZR, D, 0.0)
  for t in range(RPT // ZR):
    pltpu.async_copy(zrow, acc_sh.at[pl.ds(s * RPT + t * ZR, ZR)], semz)
  for t in range(RPT // ZR):
    pltpu.make_async_copy(zrow, acc_sh.at[pl.ds(s * RPT, ZR)], semz).wait()
  plsc.subcore_barrier()

  x_ref = xy_hbm.at[c]

  def gath(j, b):
    pltpu.async_copy(x_ref.at[gi_grp.at[j, 0]], rows[b], semg[b])

  def wait_g(b):
    pltpu.make_async_copy(x_ref.at[gi_grp.at[0, 0]], rows[b], semg[b]).wait()

  def scat(j, b):
    pltpu.async_copy(rows[b], acc_sh.at[si_grp.at[j, 0]], sems[b], add=True)

  def wait_s(b):
    pltpu.make_async_copy(rows[b], acc_sh.at[si_grp.at[0, 0]], sems[b]).wait()

  def group(grp, carry):
    base = s * nchunk + grp * GRP
    pltpu.sync_copy(g2_hbm.at[dirn, pl.ds(base, GRP)], gi_grp)
    pltpu.sync_copy(g2_hbm.at[1 - dirn, pl.ds(base, GRP)], si_grp)
    gath(0, 0)
    for j in range(1, GRP):
      if j >= 2:
        wait_s(j % 2)
      gath(j, j % 2)
      wait_g((j - 1) % 2)
      scat(j - 1, (j - 1) % 2)
    wait_g((GRP - 1) % 2)
    scat(GRP - 1, (GRP - 1) % 2)
    wait_s(0)
    wait_s(1)
    return carry

  lax.fori_loop(0, nchunk // GRP, group, 0)

  plsc.subcore_barrier()
  pltpu.sync_copy(acc_sh.at[pl.ds(s * RPT, RPT)],
                  oxy_hbm.at[c, pl.ds(s * RPT, RPT)])


def _sc_degrees(g2, nchunk):
  return pl.kernel(
      functools.partial(_deg_body, nchunk),
      out_type=jax.ShapeDtypeStruct((NC, N_PAD), jnp.float32),
      mesh=_mesh(),
      scratch_types=[
          pltpu.VMEM((nchunk, 1, C), jnp.int32),
          pltpu.VMEM((C,), jnp.float32),
          pltpu.VMEM((RPT,), jnp.float32),
          pltpu.VMEM_SHARED((N_PAD,), jnp.float32),
          pltpu.SemaphoreType.DMA,
          pltpu.SemaphoreType.DMA,
      ],
  )(g2)


def _sc_scatter(xy, g2, nchunk, dirn):
  return pl.kernel(
      functools.partial(_scat_body, nchunk, dirn),
      out_type=jax.ShapeDtypeStruct((NC, N_PAD, D), jnp.float32),
      mesh=_mesh(),
      scratch_types=[
          pltpu.VMEM((GRP, 1, C), jnp.int32),
          pltpu.VMEM((GRP, 1, C), jnp.int32),
      ]
      + [pltpu.VMEM((C, D), jnp.float32)] * 2
      + [
          pltpu.VMEM((ZR, D), jnp.float32),
          pltpu.VMEM_SHARED((N_PAD, D), jnp.float32),
      ]
      + [pltpu.SemaphoreType.DMA] * 5,
  )(xy, g2)


BR = 1024
GRID = N_PAD // BR


def _scale_body(deg_ref, q_ref, p_ref, xy_ref):
  r = lax.rsqrt(jnp.maximum(deg_ref[...], 1.0))
  xy_ref[0] = q_ref[...] * r
  xy_ref[1] = p_ref[...] * r


def _dense_body(deg_ref, axy_ref, Wq_, Wp_, W1a_, W1b_, W2_, WqT_,
                WpT_, W1aT_, W1bT_, W2T_, bq_, bp_, b1_, b2_, bxy_ref):
  r = lax.rsqrt(jnp.maximum(deg_ref[...], 1.0))
  dot = functools.partial(jnp.dot, preferred_element_type=jnp.float32)
  qg = dot(axy_ref[0] * r, Wq_[...]) + bq_[...]
  pg = dot(axy_ref[1] * r, Wp_[...]) + bp_[...]
  h1 = jnp.tanh(dot(qg, W1a_[...]) + dot(pg, W1b_[...]) + b1_[...])
  h2 = dot(h1, W2_[...]) + b2_[...]
  G = dot(h2, W2T_[...]) * (1.0 - h1 * h1)
  bxy_ref[0] = dot(dot(G, W1aT_[...]), WqT_[...]) * r
  bxy_ref[1] = dot(dot(G, W1bT_[...]), WpT_[...]) * r


def _final_body(deg_ref, q_ref, p_ref, cxy_ref, qn_ref, pn_ref):
  r = lax.rsqrt(jnp.maximum(deg_ref[...], 1.0))
  qn_ref[...] = q_ref[...] + cxy_ref[1] * r
  pn_ref[...] = p_ref[...] - cxy_ref[0] * r


def _col_spec():
  return pl.BlockSpec((BR, 1), lambda i: (i, 0))


def _row_spec():
  return pl.BlockSpec((BR, D), lambda i: (i, 0))


def _stk_spec():
  return pl.BlockSpec((2, BR, D), lambda i: (0, i, 0))


def _w_spec(shape):
  return pl.BlockSpec(shape, lambda i: (0, 0))


def kernel(g, q, p, create_graph, Wq, bq, Wp, bp, W1, b1, W2, b2):
  del create_graph
  E = g.shape[1]
  et_raw = -(-E // NS)
  nchunk_raw = -(-et_raw // C)
  nchunk = -(-nchunk_raw // GRP) * GRP
  e_pad = nchunk * C * NS
  src = g[0].astype(jnp.int32)
  dst = g[1].astype(jnp.int32)
  npad_idx = e_pad - E
  if npad_idx:
    pad_idx = N + jnp.arange(npad_idx, dtype=jnp.int32) % (N_PAD - N)
    src = jnp.concatenate([src, pad_idx])
    dst = jnp.concatenate([dst, pad_idx])
  g2 = jnp.stack([src, dst]).reshape(NC, NS * nchunk, 1, C)

  degs = _sc_degrees(g2, nchunk)
  out_deg = degs[0].reshape(N_PAD, 1)
  in_deg = degs[1].reshape(N_PAD, 1)

  qp = pl.pallas_call(
      _scale_body,
      grid=(GRID,),
      in_specs=[_col_spec(), _row_spec(), _row_spec()],
      out_specs=_stk_spec(),
      out_shape=jax.ShapeDtypeStruct((2, N_PAD, D), jnp.float32),
  )(out_deg, q, p)

  agg = _sc_scatter(qp, g2, nchunk, 0)

  W1a, W1b = W1[:D], W1[D:]
  B = pl.pallas_call(
      _dense_body,
      grid=(GRID,),
      in_specs=[_col_spec(), _stk_spec()]
      + [_w_spec((D, D))] * 10
      + [_w_spec((1, D))] * 4,
      out_specs=_stk_spec(),
      out_shape=jax.ShapeDtypeStruct((2, N_PAD, D), jnp.float32),
  )(in_deg, agg, Wq, Wp, W1a, W1b, W2, Wq.T, Wp.T, W1a.T, W1b.T, W2.T,
    bq.reshape(1, D), bp.reshape(1, D), b1.reshape(1, D), b2.reshape(1, D))

  Cqp = _sc_scatter(B, g2, nchunk, 1)

  q_next, p_next = pl.pallas_call(
      _final_body,
      grid=(GRID,),
      in_specs=[_col_spec(), _row_spec(), _row_spec(), _stk_spec()],
      out_specs=[_row_spec(), _row_spec()],
      out_shape=[jax.ShapeDtypeStruct((N, D), jnp.float32)] * 2,
  )(out_deg, q, p, Cqp)

  return (q_next, p_next)


# speedup vs baseline: 11.8844x; 1.0060x over previous
import functools

import jax
import jax.numpy as jnp
from jax import lax
from jax.experimental import pallas as pl
from jax.experimental.pallas import tpu as pltpu
from jax.experimental.pallas import tpu_sc as plsc

N = 10000
D = 128
NC = 2
NS = 16
LANES = 16
C = 128
GRP = 40
N_PAD = 10240
RPT = N_PAD // NS
N_ACC = 10112
RPA = N_ACC // NS
DEG_GRP = 8


@functools.cache
def _mesh():
  return plsc.VectorSubcoreMesh(
      core_axis_name="c", subcore_axis_name="s", num_cores=NC, num_subcores=NS)


def _fill(ref, rows, cols, value):
  v = jnp.full((LANES,), value, jnp.float32)
  for r in range(rows):
    for j in range(cols // LANES):
      if rows == 1:
        ref[pl.ds(j * LANES, LANES)] = v
      else:
        ref[r, pl.ds(j * LANES, LANES)] = v


def _deg_body(nchunk, g2_hbm, deg_hbm, idx_big, ones_v, zb_v, sh_deg,
              semz, semsc):
  c = lax.axis_index("c")
  s = lax.axis_index("s")
  _fill(ones_v, 1, C, 1.0)
  _fill(zb_v, 1, RPT, 0.0)
  pltpu.async_copy(zb_v, sh_deg.at[pl.ds(s * RPT, RPT)], semz)
  pltpu.sync_copy(g2_hbm.at[c, pl.ds(s * nchunk, nchunk)], idx_big)
  pltpu.make_async_copy(zb_v, sh_deg.at[pl.ds(s * RPT, RPT)], semz).wait()
  plsc.subcore_barrier()

  def fire(t):
    for b in range(DEG_GRP):
      pltpu.async_copy(ones_v, sh_deg.at[idx_big.at[t * DEG_GRP + b, 0]],
                       semsc, add=True)

  def drain():
    for _ in range(DEG_GRP):
      pltpu.make_async_copy(ones_v, sh_deg.at[idx_big.at[0, 0]], semsc).wait()

  fire(0)

  def body(t, carry):
    fire(t)
    drain()
    return carry

  lax.fori_loop(1, nchunk // DEG_GRP, body, 0)
  drain()
  plsc.subcore_barrier()
  pltpu.sync_copy(sh_deg.at[pl.ds(s * RPT, RPT)],
                  deg_hbm.at[c, pl.ds(s * RPT, RPT)])


def _scat_body(nchunk, dirn, xy_hbm, g2_hbm, z_hbm, oxy_hbm,
               gi_grp, si_grp, r0, r1, acc_sh,
               semz, semi, sg0, sg1, ss0, ss1):
  rows = [r0, r1]
  semg = [sg0, sg1]
  sems = [ss0, ss1]
  c = lax.axis_index("c")
  s = lax.axis_index("s")
  ngrp = nchunk // GRP

  def load_idx(grp):
    base = s * nchunk + grp * GRP
    pltpu.async_copy(g2_hbm.at[dirn, pl.ds(base, GRP)], gi_grp, semi)
    pltpu.async_copy(g2_hbm.at[1 - dirn, pl.ds(base, GRP)], si_grp, semi)

  def wait_idx():
    pltpu.make_async_copy(g2_hbm.at[dirn, pl.ds(0, GRP)], gi_grp, semi).wait()
    pltpu.make_async_copy(g2_hbm.at[dirn, pl.ds(0, GRP)], si_grp, semi).wait()

  pltpu.async_copy(z_hbm.at[pl.ds(s * RPA, RPA)],
                   acc_sh.at[pl.ds(s * RPA, RPA)], semz)
  load_idx(0)
  pltpu.make_async_copy(z_hbm.at[pl.ds(s * RPA, RPA)],
                        acc_sh.at[pl.ds(s * RPA, RPA)], semz).wait()
  plsc.subcore_barrier()

  x_ref = xy_hbm.at[c]

  def gath(j, b):
    pltpu.async_copy(x_ref.at[gi_grp.at[j, 0]], rows[b], semg[b])

  def wait_g(b):
    pltpu.make_async_copy(x_ref.at[gi_grp.at[0, 0]], rows[b], semg[b]).wait()

  def scat(j, b):
    pltpu.async_copy(rows[b], acc_sh.at[si_grp.at[j, 0]], sems[b], add=True)

  def wait_s(b):
    pltpu.make_async_copy(rows[b], acc_sh.at[si_grp.at[0, 0]], sems[b]).wait()

  def group(grp, carry):
    wait_idx()
    gath(0, 0)
    for j in range(1, GRP):
      if j >= 2:
        wait_s(j % 2)
      gath(j, j % 2)
      wait_g((j - 1) % 2)
      scat(j - 1, (j - 1) % 2)
    wait_g((GRP - 1) % 2)
    scat(GRP - 1, (GRP - 1) % 2)
    wait_s(0)
    wait_s(1)
    load_idx(lax.rem(grp + 1, ngrp))
    return carry

  lax.fori_loop(0, ngrp, group, 0)
  wait_idx()

  plsc.subcore_barrier()
  pltpu.sync_copy(acc_sh.at[pl.ds(s * RPA, RPA)],
                  oxy_hbm.at[c, pl.ds(s * RPA, RPA)])


def _sc_degrees(g2, nchunk):
  return pl.kernel(
      functools.partial(_deg_body, nchunk),
      out_type=jax.ShapeDtypeStruct((NC, N_PAD), jnp.float32),
      mesh=_mesh(),
      scratch_types=[
          pltpu.VMEM((nchunk, 1, C), jnp.int32),
          pltpu.VMEM((C,), jnp.float32),
          pltpu.VMEM((RPT,), jnp.float32),
          pltpu.VMEM_SHARED((N_PAD,), jnp.float32),
          pltpu.SemaphoreType.DMA,
          pltpu.SemaphoreType.DMA,
      ],
  )(g2)


def _sc_scatter(xy, g2, zeros, nchunk, dirn):
  return pl.kernel(
      functools.partial(_scat_body, nchunk, dirn),
      out_type=jax.ShapeDtypeStruct((NC, N_ACC, D), jnp.float32),
      mesh=_mesh(),
      scratch_types=[
          pltpu.VMEM((GRP, 1, C), jnp.int32),
          pltpu.VMEM((GRP, 1, C), jnp.int32),
      ]
      + [pltpu.VMEM((C, D), jnp.float32)] * 2
      + [pltpu.VMEM_SHARED((N_ACC, D), jnp.float32)]
      + [pltpu.SemaphoreType.DMA] * 6,
  )(xy, g2, zeros)


BR = 1024
GRID = N_PAD // BR


def _scale_body(deg_ref, q_ref, p_ref, xy_ref):
  r = lax.rsqrt(jnp.maximum(deg_ref[...], 1.0))
  xy_ref[0] = q_ref[...] * r
  xy_ref[1] = p_ref[...] * r


def _dense_body(deg_ref, axy_ref, Wq_, Wp_, W1a_, W1b_, W2_, WqT_,
                WpT_, W1aT_, W1bT_, W2T_, bq_, bp_, b1_, b2_, bxy_ref):
  r = lax.rsqrt(jnp.maximum(deg_ref[...], 1.0))
  dot = functools.partial(jnp.dot, preferred_element_type=jnp.float32)
  qg = dot(axy_ref[0] * r, Wq_[...]) + bq_[...]
  pg = dot(axy_ref[1] * r, Wp_[...]) + bp_[...]
  h1 = jnp.tanh(dot(qg, W1a_[...]) + dot(pg, W1b_[...]) + b1_[...])
  h2 = dot(h1, W2_[...]) + b2_[...]
  G = dot(h2, W2T_[...]) * (1.0 - h1 * h1)
  bxy_ref[0] = dot(dot(G, W1aT_[...]), WqT_[...]) * r
  bxy_ref[1] = dot(dot(G, W1bT_[...]), WpT_[...]) * r


def _final_body(deg_ref, q_ref, p_ref, cxy_ref, qn_ref, pn_ref):
  r = lax.rsqrt(jnp.maximum(deg_ref[...], 1.0))
  qn_ref[...] = q_ref[...] + cxy_ref[1] * r
  pn_ref[...] = p_ref[...] - cxy_ref[0] * r


def _col_spec():
  return pl.BlockSpec((BR, 1), lambda i: (i, 0))


def _row_spec():
  return pl.BlockSpec((BR, D), lambda i: (i, 0))


def _stk_spec():
  return pl.BlockSpec((2, BR, D), lambda i: (0, i, 0))


def _w_spec(shape):
  return pl.BlockSpec(shape, lambda i: (0, 0))


def kernel(g, q, p, create_graph, Wq, bq, Wp, bp, W1, b1, W2, b2):
  del create_graph
  E = g.shape[1]
  et_raw = -(-E // NS)
  nchunk_raw = -(-et_raw // C)
  nchunk = -(-nchunk_raw // GRP) * GRP
  e_pad = nchunk * C * NS
  src = g[0].astype(jnp.int32)
  dst = g[1].astype(jnp.int32)
  npad_idx = e_pad - E
  if npad_idx:
    pad_idx = N + jnp.arange(npad_idx, dtype=jnp.int32) % (N_ACC - N)
    src = jnp.concatenate([src, pad_idx])
    dst = jnp.concatenate([dst, pad_idx])
  g2 = jnp.stack([src, dst]).reshape(NC, NS * nchunk, 1, C)
  zeros = jnp.zeros((N_ACC, D), jnp.float32)

  degs = _sc_degrees(g2, nchunk)
  out_deg = degs[0].reshape(N_PAD, 1)
  in_deg = degs[1].reshape(N_PAD, 1)

  qp = pl.pallas_call(
      _scale_body,
      grid=(GRID,),
      in_specs=[_col_spec(), _row_spec(), _row_spec()],
      out_specs=_stk_spec(),
      out_shape=jax.ShapeDtypeStruct((2, N_ACC, D), jnp.float32),
  )(out_deg, q, p)

  agg = _sc_scatter(qp, g2, zeros, nchunk, 0)

  W1a, W1b = W1[:D], W1[D:]
  B = pl.pallas_call(
      _dense_body,
      grid=(GRID,),
      in_specs=[_col_spec(), _stk_spec()]
      + [_w_spec((D, D))] * 10
      + [_w_spec((1, D))] * 4,
      out_specs=_stk_spec(),
      out_shape=jax.ShapeDtypeStruct((2, N_ACC, D), jnp.float32),
  )(in_deg, agg, Wq, Wp, W1a, W1b, W2, Wq.T, Wp.T, W1a.T, W1b.T, W2.T,
    bq.reshape(1, D), bp.reshape(1, D), b1.reshape(1, D), b2.reshape(1, D))

  Cqp = _sc_scatter(B, g2, zeros, nchunk, 1)

  q_next, p_next = pl.pallas_call(
      _final_body,
      grid=(GRID,),
      in_specs=[_col_spec(), _row_spec(), _row_spec(), _stk_spec()],
      out_specs=[_row_spec(), _row_spec()],
      out_shape=[jax.ShapeDtypeStruct((N, D), jnp.float32)] * 2,
  )(out_deg, q, p, Cqp)

  return (q_next, p_next)
